# baseline (speedup 1.0000x reference)
import jax
import jax.numpy as jnp
from jax.experimental import pallas as pl
from jax.experimental.pallas import tpu as pltpu

_N = 4096
_D = 32
_DA = _D + 2
_K = 3
_BLK = 256


def _lmnn_kernel(x_blk_ref, x_full_ref, lab_blk_ref, lab_full_ref,
                 out_ref, pull_acc, push_acc, imp_acc,
                 d_ref, aug_ref):
    i = pl.program_id(0)
    nblk = pl.num_programs(0)
    inf = jnp.bfloat16(jnp.inf)

    @pl.when(i == 0)
    def _init():
        pull_acc[0] = 0.0
        push_acc[0] = 0.0
        imp_acc[0] = 0
        x_full = x_full_ref[...]
        a2f = jnp.sum(x_full * x_full, axis=1, keepdims=True)
        aug_ref[...] = jnp.concatenate(
            [x_full, jnp.ones((_N, 1), jnp.float32), a2f],
            axis=1).astype(jnp.bfloat16)

    x_blk = x_blk_ref[...]
    lab_blk = lab_blk_ref[...]
    lab_full = lab_full_ref[...]

    a2b = jnp.sum(x_blk * x_blk, axis=1, keepdims=True)
    aug_b = jnp.concatenate(
        [x_blk * -2.0, a2b, jnp.ones((_BLK, 1), jnp.float32)],
        axis=1).astype(jnp.bfloat16)
    g = jax.lax.dot_general(
        aug_b, aug_ref[...], (((1,), (1,)), ((), ())),
        preferred_element_type=jnp.float32)
    d_ref[...] = jnp.maximum(g, 0.0).astype(jnp.bfloat16)

    eye = (jax.lax.broadcasted_iota(jnp.int16, (_BLK, _BLK), 0)
           == jax.lax.broadcasted_iota(jnp.int16, (_BLK, _BLK), 1))
    dslice = d_ref[:, pl.ds(i * _BLK, _BLK)]
    d_ref[:, pl.ds(i * _BLK, _BLK)] = jnp.where(eye, inf, dslice)

    d = d_ref[...]
    same = lab_blk == lab_full
    dd = jnp.where(same, d, inf)

    one = jnp.bfloat16(1.0)
    zero = jnp.bfloat16(0.0)
    i16_1 = jnp.int16(1)
    i16_0 = jnp.int16(0)
    m1 = jnp.min(dd, axis=1, keepdims=True)
    e1 = dd == m1
    c1 = jnp.sum(jnp.where(e1, i16_1, i16_0), axis=1, keepdims=True)
    m2 = jnp.min(jnp.where(e1, inf, dd), axis=1, keepdims=True)
    le = dd <= m2
    c12 = jnp.sum(jnp.where(le, i16_1, i16_0), axis=1, keepdims=True)
    m3 = jnp.min(jnp.where(le, inf, dd), axis=1, keepdims=True)

    k2 = jnp.where(c1 >= jnp.int16(2), m1, m2)
    k3 = jnp.where(c1 >= jnp.int16(3), m1,
                   jnp.where(c12 >= jnp.int16(3), m2, m3))

    margin_v = one + k3
    sum3 = (m1.astype(jnp.float32) + k2.astype(jnp.float32)
            + k3.astype(jnp.float32))
    margin_f = margin_v.astype(jnp.float32)
    corr_v = 3.0 * margin_f - sum3

    t = margin_v - d
    relu_t = jnp.maximum(t, zero)
    s_all = jnp.sum(relu_t.astype(jnp.float32), axis=1, keepdims=True)
    imp_rows = jnp.max(jnp.where(same, -inf, t), axis=1, keepdims=True)
    imp = jnp.max(imp_rows.astype(jnp.float32)) > 0.0

    pull_b = jnp.sum(sum3)
    push_b = jnp.sum(s_all - corr_v)
    imp_b = imp.astype(jnp.int32)

    deg = jnp.max(margin_f) == jnp.float32(jnp.inf)

    @pl.when(deg)
    def _slow():
        finf = jnp.float32(jnp.inf)
        dun = jnp.maximum(g, 0.0)
        col = jax.lax.broadcasted_iota(jnp.int32, (_BLK, _N), 1)
        rowl = jax.lax.broadcasted_iota(jnp.int32, (_BLK, _N), 0)
        row_g = rowl + i * _BLK
        row_id = row_g[:, :1]
        labb = lab_blk.astype(jnp.float32)
        labf = lab_full.astype(jnp.float32)
        same2 = labb == labf
        offd = col != row_g
        ddx = jnp.where(same2 & offd, dun, finf)
        colx = col
        pull = jnp.zeros((_BLK, 1), jnp.float32)
        tmax = jnp.full((_BLK, 1), -finf)
        tvals = []
        isdiags = []
        for _ in range(_K):
            m = jnp.min(ddx, axis=1, keepdims=True)
            idx = jnp.min(jnp.where(ddx == m, colx, _N), axis=1,
                          keepdims=True)
            hit = col == idx
            tval = jnp.sum(jnp.where(hit, dun, 0.0), axis=1,
                           keepdims=True)
            ddx = jnp.where(hit, finf, ddx)
            colx = jnp.where(hit, _N, colx)
            pull = pull + tval
            tmax = jnp.maximum(tmax, tval)
            tvals.append(tval)
            isdiags.append(idx == row_id)
        margin_sf = 1.0 + tmax
        corr_s = jnp.zeros((_BLK, 1), jnp.float32)
        for tval, isdiag in zip(tvals, isdiags):
            corr_s = corr_s + jnp.where(
                isdiag, 0.0, jnp.maximum(margin_sf - tval, 0.0))
        dp = jnp.where(offd, dun, finf)
        ts = margin_sf - dp
        s_all_s = jnp.sum(jnp.maximum(ts, 0.0), axis=1, keepdims=True)
        imp_s = jnp.max(jnp.where(same2, -finf, ts)) > 0.0
        pull_acc[0] = pull_acc[0] + jnp.sum(pull)
        push_acc[0] = push_acc[0] + jnp.sum(s_all_s - corr_s)
        imp_acc[0] = imp_acc[0] | imp_s.astype(jnp.int32)

    @pl.when(jnp.logical_not(deg))
    def _fast():
        pull_acc[0] = pull_acc[0] + pull_b
        push_acc[0] = push_acc[0] + push_b
        imp_acc[0] = imp_acc[0] | imp_b

    @pl.when(i == nblk - 1)
    def _fin():
        p = pull_acc[0]
        s = push_acc[0]
        total = jnp.where(imp_acc[0] > 0, (p + s) / _N, p / _N)
        out_ref[...] = jnp.broadcast_to(total, (1, 1))


def kernel(outputs, label_inds):
    lab = label_inds.astype(jnp.bfloat16)
    lab_col = lab.reshape(_N, 1)
    lab_row = lab.reshape(1, _N)
    grid = _N // _BLK
    out = pl.pallas_call(
        _lmnn_kernel,
        grid=(grid,),
        in_specs=[
            pl.BlockSpec((_BLK, _D), lambda i: (i, 0)),
            pl.BlockSpec((_N, _D), lambda i: (0, 0)),
            pl.BlockSpec((_BLK, 1), lambda i: (i, 0)),
            pl.BlockSpec((1, _N), lambda i: (0, 0)),
        ],
        out_specs=pl.BlockSpec((1, 1), lambda i: (0, 0)),
        out_shape=jax.ShapeDtypeStruct((1, 1), jnp.float32),
        scratch_shapes=[
            pltpu.SMEM((1,), jnp.float32),
            pltpu.SMEM((1,), jnp.float32),
            pltpu.SMEM((1,), jnp.int32),
            pltpu.VMEM((_BLK, _N), jnp.bfloat16),
            pltpu.VMEM((_N, _DA), jnp.bfloat16),
        ],
        compiler_params=pltpu.CompilerParams(
            dimension_semantics=("arbitrary",)),
    )(outputs, outputs, lab_col, lab_row)
    return out[0, 0]


# speedup vs baseline: 1.0502x; 1.0502x over previous
import jax
import jax.numpy as jnp
from jax.experimental import pallas as pl
from jax.experimental.pallas import tpu as pltpu

_N = 4096
_D = 32
_DA = _D + 2
_K = 3
_BLK = 512


def _lmnn_kernel(x_blk_ref, x_full_ref, lab_blk_ref, lab_full_ref,
                 out_ref, pull_acc, push_acc, imp_acc,
                 d_ref, aug_ref):
    i = pl.program_id(0)
    nblk = pl.num_programs(0)
    inf = jnp.bfloat16(jnp.inf)

    @pl.when(i == 0)
    def _init():
        pull_acc[0] = 0.0
        push_acc[0] = 0.0
        imp_acc[0] = 0
        x_full = x_full_ref[...]
        a2f = jnp.sum(x_full * x_full, axis=1, keepdims=True)
        aug_ref[...] = jnp.concatenate(
            [x_full, jnp.ones((_N, 1), jnp.float32), a2f],
            axis=1).astype(jnp.bfloat16)

    x_blk = x_blk_ref[...]
    lab_blk = lab_blk_ref[...]
    lab_full = lab_full_ref[...]

    a2b = jnp.sum(x_blk * x_blk, axis=1, keepdims=True)
    aug_b = jnp.concatenate(
        [x_blk * -2.0, a2b, jnp.ones((_BLK, 1), jnp.float32)],
        axis=1).astype(jnp.bfloat16)
    g = jax.lax.dot_general(
        aug_b, aug_ref[...], (((1,), (1,)), ((), ())),
        preferred_element_type=jnp.float32)
    d_ref[...] = jnp.maximum(g, 0.0).astype(jnp.bfloat16)

    eye = (jax.lax.broadcasted_iota(jnp.int16, (_BLK, _BLK), 0)
           == jax.lax.broadcasted_iota(jnp.int16, (_BLK, _BLK), 1))
    dslice = d_ref[:, pl.ds(i * _BLK, _BLK)]
    d_ref[:, pl.ds(i * _BLK, _BLK)] = jnp.where(eye, inf, dslice)

    d = d_ref[...]
    same = lab_blk == lab_full
    dd = jnp.where(same, d, inf)

    one = jnp.bfloat16(1.0)
    zero = jnp.bfloat16(0.0)
    i16_1 = jnp.int16(1)
    i16_0 = jnp.int16(0)
    m1 = jnp.min(dd, axis=1, keepdims=True)
    e1 = dd == m1
    c1 = jnp.sum(jnp.where(e1, i16_1, i16_0), axis=1, keepdims=True)
    m2 = jnp.min(jnp.where(e1, inf, dd), axis=1, keepdims=True)
    le = dd <= m2
    c12 = jnp.sum(jnp.where(le, i16_1, i16_0), axis=1, keepdims=True)
    m3 = jnp.min(jnp.where(le, inf, dd), axis=1, keepdims=True)

    k2 = jnp.where(c1 >= jnp.int16(2), m1, m2)
    k3 = jnp.where(c1 >= jnp.int16(3), m1,
                   jnp.where(c12 >= jnp.int16(3), m2, m3))

    margin_v = one + k3
    sum3 = (m1.astype(jnp.float32) + k2.astype(jnp.float32)
            + k3.astype(jnp.float32))
    margin_f = margin_v.astype(jnp.float32)
    corr_v = 3.0 * margin_f - sum3

    t = margin_v - d
    relu_t = jnp.maximum(t, zero)
    s_all = jnp.sum(relu_t.astype(jnp.float32), axis=1, keepdims=True)
    imp_rows = jnp.max(jnp.where(same, -inf, t), axis=1, keepdims=True)
    imp = jnp.max(imp_rows.astype(jnp.float32)) > 0.0

    pull_b = jnp.sum(sum3)
    push_b = jnp.sum(s_all - corr_v)
    imp_b = imp.astype(jnp.int32)

    deg = jnp.max(margin_f) == jnp.float32(jnp.inf)

    @pl.when(deg)
    def _slow():
        finf = jnp.float32(jnp.inf)
        dun = jnp.maximum(g, 0.0)
        col = jax.lax.broadcasted_iota(jnp.int32, (_BLK, _N), 1)
        rowl = jax.lax.broadcasted_iota(jnp.int32, (_BLK, _N), 0)
        row_g = rowl + i * _BLK
        row_id = row_g[:, :1]
        labb = lab_blk.astype(jnp.float32)
        labf = lab_full.astype(jnp.float32)
        same2 = labb == labf
        offd = col != row_g
        ddx = jnp.where(same2 & offd, dun, finf)
        colx = col
        pull = jnp.zeros((_BLK, 1), jnp.float32)
        tmax = jnp.full((_BLK, 1), -finf)
        tvals = []
        isdiags = []
        for _ in range(_K):
            m = jnp.min(ddx, axis=1, keepdims=True)
            idx = jnp.min(jnp.where(ddx == m, colx, _N), axis=1,
                          keepdims=True)
            hit = col == idx
            tval = jnp.sum(jnp.where(hit, dun, 0.0), axis=1,
                           keepdims=True)
            ddx = jnp.where(hit, finf, ddx)
            colx = jnp.where(hit, _N, colx)
            pull = pull + tval
            tmax = jnp.maximum(tmax, tval)
            tvals.append(tval)
            isdiags.append(idx == row_id)
        margin_sf = 1.0 + tmax
        corr_s = jnp.zeros((_BLK, 1), jnp.float32)
        for tval, isdiag in zip(tvals, isdiags):
            corr_s = corr_s + jnp.where(
                isdiag, 0.0, jnp.maximum(margin_sf - tval, 0.0))
        dp = jnp.where(offd, dun, finf)
        ts = margin_sf - dp
        s_all_s = jnp.sum(jnp.maximum(ts, 0.0), axis=1, keepdims=True)
        imp_s = jnp.max(jnp.where(same2, -finf, ts)) > 0.0
        pull_acc[0] = pull_acc[0] + jnp.sum(pull)
        push_acc[0] = push_acc[0] + jnp.sum(s_all_s - corr_s)
        imp_acc[0] = imp_acc[0] | imp_s.astype(jnp.int32)

    @pl.when(jnp.logical_not(deg))
    def _fast():
        pull_acc[0] = pull_acc[0] + pull_b
        push_acc[0] = push_acc[0] + push_b
        imp_acc[0] = imp_acc[0] | imp_b

    @pl.when(i == nblk - 1)
    def _fin():
        p = pull_acc[0]
        s = push_acc[0]
        total = jnp.where(imp_acc[0] > 0, (p + s) / _N, p / _N)
        out_ref[...] = jnp.broadcast_to(total, (1, 1))


def kernel(outputs, label_inds):
    lab = label_inds.astype(jnp.bfloat16)
    lab_col = lab.reshape(_N, 1)
    lab_row = lab.reshape(1, _N)
    grid = _N // _BLK
    out = pl.pallas_call(
        _lmnn_kernel,
        grid=(grid,),
        in_specs=[
            pl.BlockSpec((_BLK, _D), lambda i: (i, 0)),
            pl.BlockSpec((_N, _D), lambda i: (0, 0)),
            pl.BlockSpec((_BLK, 1), lambda i: (i, 0)),
            pl.BlockSpec((1, _N), lambda i: (0, 0)),
        ],
        out_specs=pl.BlockSpec((1, 1), lambda i: (0, 0)),
        out_shape=jax.ShapeDtypeStruct((1, 1), jnp.float32),
        scratch_shapes=[
            pltpu.SMEM((1,), jnp.float32),
            pltpu.SMEM((1,), jnp.float32),
            pltpu.SMEM((1,), jnp.int32),
            pltpu.VMEM((_BLK, _N), jnp.bfloat16),
            pltpu.VMEM((_N, _DA), jnp.bfloat16),
        ],
        compiler_params=pltpu.CompilerParams(
            dimension_semantics=("arbitrary",)),
    )(outputs, outputs, lab_col, lab_row)
    return out[0, 0]
